# baseline (speedup 1.0000x reference)
import jax
import jax.numpy as jnp
from jax.experimental import pallas as pl

B, T, N = 64, 12, 325
H, C = 2, 64
LSTM_U = 16
OUT_STEPS, NUM_FEATURES = 12, 325
D_OUT = OUT_STEPS * NUM_FEATURES
D_IN = N * C + LSTM_U

_NB = 16
_JBLK = 256
_NJB = (D_OUT + _JBLK - 1) // _JBLK


def _gat_body(x3_ref, at_ref, k2t_ref, ksn_ref, ksnt_ref, bias_ref, g_ref):
    f32 = jnp.float32
    at_blk = at_ref[:]
    bias_col = bias_ref[:]
    x3 = x3_ref[:]
    for b in range(_NB):
        x_b = x3[:, b, :]
        att_rows = jnp.dot(ksnt_ref[:], x_b, preferred_element_type=f32)
        att_cols = jax.lax.dot_general(
            x_b, ksn_ref[:], (((0,), (0,)), ((), ())),
            preferred_element_type=f32)
        fT = jnp.dot(k2t_ref[:], x_b, preferred_element_type=f32)
        acc = jnp.zeros((C, N), f32)
        for h in range(H):
            lgT = att_cols[:, 2 + h:3 + h] + att_rows[h:h + 1, :]
            lgT = jnp.where(lgT >= 0.0, lgT, 0.2 * lgT)
            e = jnp.exp(lgT) * at_blk
            s = jnp.sum(e, axis=0, keepdims=True)
            outT = jnp.dot(fT[h * C:(h + 1) * C, :], e,
                           preferred_element_type=f32)
            acc = acc + outT / s
        gT = jnp.maximum(acc * (1.0 / H) + bias_col, 0.0)
        g_ref[b] = gT.astype(jnp.bfloat16)


def _lstm_body(xflat_ref, w_ref, u_ref, b_ref, h_ref):
    f32 = jnp.float32
    xz = jax.lax.dot_general(xflat_ref[:], w_ref[:], (((1,), (1,)), ((), ())),
                             preferred_element_type=f32)
    h = jnp.zeros((B, LSTM_U), f32)
    c = jnp.zeros((B, LSTM_U), f32)
    U4 = 4 * LSTM_U
    for t in range(T):
        z = xz[t * B:(t + 1) * B, :] + jnp.dot(
            h, u_ref[:], preferred_element_type=f32) + b_ref[:]
        i = jax.nn.sigmoid(z[:, 0 * LSTM_U:1 * LSTM_U])
        fg = jax.nn.sigmoid(z[:, 1 * LSTM_U:2 * LSTM_U])
        g = jnp.tanh(z[:, 2 * LSTM_U:3 * LSTM_U])
        o = jax.nn.sigmoid(z[:, 3 * LSTM_U:U4])
        c = fg * c + i * g
        h = o * jnp.tanh(c)
    h_ref[:] = h.astype(jnp.bfloat16)


def _dense_body(z_ref, wt0_ref, wt1_ref, b_ref, out_ref):
    z = z_ref[:]
    half = _JBLK // 2
    for i, wt_ref in enumerate((wt0_ref, wt1_ref)):
        wb = wt_ref[:].astype(jnp.bfloat16)
        part = jax.lax.dot_general(
            z, wb, (((1,), (1,)), ((), ())),
            preferred_element_type=jnp.float32)
        out_ref[:, i * half:(i + 1) * half] = (
            part + b_ref[:, i * half:(i + 1) * half])


def kernel(x, a, gat_kernel, attn_self, attn_neigh, gat_bias,
           lstm_W, lstm_Uw, lstm_b, dense_W, dense_b):
    f32 = jnp.float32
    x312 = jnp.transpose(x, (1, 0, 2))
    k2 = gat_kernel.reshape(T, H * C)
    hsel = jnp.repeat(jnp.eye(H, dtype=f32), C, axis=0)
    sflat = attn_self[:, :, 0].T.reshape(H * C)
    nflat = attn_neigh[:, :, 0].T.reshape(H * C)
    sn = jnp.concatenate([sflat[:, None] * hsel, nflat[:, None] * hsel], axis=1)
    ksn = k2 @ sn

    g = pl.pallas_call(
        _gat_body,
        grid=(B // _NB,),
        in_specs=[
            pl.BlockSpec((T, _NB, N), lambda b: (0, b, 0)),
            pl.BlockSpec((N, N), lambda b: (0, 0)),
            pl.BlockSpec((H * C, T), lambda b: (0, 0)),
            pl.BlockSpec((T, 4), lambda b: (0, 0)),
            pl.BlockSpec((4, T), lambda b: (0, 0)),
            pl.BlockSpec((C, 1), lambda b: (0, 0)),
        ],
        out_specs=pl.BlockSpec((_NB, C, N), lambda b: (b, 0, 0)),
        out_shape=jax.ShapeDtypeStruct((B, C, N), jnp.bfloat16),
    )(x312, a.T, k2.T, ksn, ksn.T, gat_bias.reshape(C, 1))

    xflat = x312.reshape(T * B, N)
    h = pl.pallas_call(
        _lstm_body,
        out_shape=jax.ShapeDtypeStruct((B, LSTM_U), jnp.bfloat16),
    )(xflat, lstm_W.T, lstm_Uw, lstm_b.reshape(1, 4 * LSTM_U))

    g2 = jnp.transpose(g, (0, 2, 1)).reshape(B, N * C)
    z = jnp.concatenate([g2, h], axis=1)
    wt = dense_W.T

    out = pl.pallas_call(
        _dense_body,
        grid=(_NJB,),
        in_specs=[
            pl.BlockSpec((B, D_IN), lambda j: (0, 0)),
            pl.BlockSpec((_JBLK // 2, D_IN), lambda j: (2 * j, 0)),
            pl.BlockSpec((_JBLK // 2, D_IN),
                         lambda j: (jnp.minimum(2 * j + 1, 2 * _NJB - 3), 0)),
            pl.BlockSpec((1, _JBLK), lambda j: (0, j)),
        ],
        out_specs=pl.BlockSpec((B, _JBLK), lambda j: (0, j)),
        out_shape=jax.ShapeDtypeStruct((B, D_OUT), f32),
    )(z, wt, wt, dense_b.reshape(1, D_OUT))

    return out.reshape(B, OUT_STEPS, NUM_FEATURES)


# speedup vs baseline: 1.0048x; 1.0048x over previous
import jax
import jax.numpy as jnp
from jax.experimental import pallas as pl

B, T, N = 64, 12, 325
H, C = 2, 64
LSTM_U = 16
OUT_STEPS, NUM_FEATURES = 12, 325
D_OUT = OUT_STEPS * NUM_FEATURES
D_IN = N * C + LSTM_U

_NB = 32
_JBLK = 256
_NJB = (D_OUT + _JBLK - 1) // _JBLK


def _gat_body(x3_ref, at_ref, k2t_ref, ksn_ref, ksnt_ref, bias_ref, g_ref):
    f32 = jnp.float32
    at_blk = at_ref[:]
    bias_col = bias_ref[:]
    x3 = x3_ref[:]
    for b in range(_NB):
        x_b = x3[:, b, :]
        att_rows = jnp.dot(ksnt_ref[:], x_b, preferred_element_type=f32)
        att_cols = jax.lax.dot_general(
            x_b, ksn_ref[:], (((0,), (0,)), ((), ())),
            preferred_element_type=f32)
        fT = jnp.dot(k2t_ref[:], x_b, preferred_element_type=f32)
        acc = jnp.zeros((C, N), f32)
        for h in range(H):
            lgT = att_cols[:, 2 + h:3 + h] + att_rows[h:h + 1, :]
            lgT = jnp.where(lgT >= 0.0, lgT, 0.2 * lgT)
            e = jnp.exp(lgT) * at_blk
            s = jnp.sum(e, axis=0, keepdims=True)
            outT = jnp.dot(fT[h * C:(h + 1) * C, :], e,
                           preferred_element_type=f32)
            acc = acc + outT / s
        gT = jnp.maximum(acc * (1.0 / H) + bias_col, 0.0)
        g_ref[b] = gT.astype(jnp.bfloat16)


def _lstm_body(xflat_ref, w_ref, u_ref, b_ref, h_ref):
    f32 = jnp.float32
    xz = jax.lax.dot_general(xflat_ref[:], w_ref[:], (((1,), (1,)), ((), ())),
                             preferred_element_type=f32)
    h = jnp.zeros((B, LSTM_U), f32)
    c = jnp.zeros((B, LSTM_U), f32)
    U4 = 4 * LSTM_U
    for t in range(T):
        z = xz[t * B:(t + 1) * B, :] + jnp.dot(
            h, u_ref[:], preferred_element_type=f32) + b_ref[:]
        i = jax.nn.sigmoid(z[:, 0 * LSTM_U:1 * LSTM_U])
        fg = jax.nn.sigmoid(z[:, 1 * LSTM_U:2 * LSTM_U])
        g = jnp.tanh(z[:, 2 * LSTM_U:3 * LSTM_U])
        o = jax.nn.sigmoid(z[:, 3 * LSTM_U:U4])
        c = fg * c + i * g
        h = o * jnp.tanh(c)
    h_ref[:] = h.astype(jnp.bfloat16)


def _dense_body(z_ref, wt_ref, b_ref, out_ref):
    wb = wt_ref[:].astype(jnp.bfloat16)
    out_ref[:] = jax.lax.dot_general(
        z_ref[:], wb, (((1,), (1,)), ((), ())),
        preferred_element_type=jnp.float32) + b_ref[:]


def kernel(x, a, gat_kernel, attn_self, attn_neigh, gat_bias,
           lstm_W, lstm_Uw, lstm_b, dense_W, dense_b):
    f32 = jnp.float32
    x312 = jnp.transpose(x, (1, 0, 2))
    k2 = gat_kernel.reshape(T, H * C)
    hsel = jnp.repeat(jnp.eye(H, dtype=f32), C, axis=0)
    sflat = attn_self[:, :, 0].T.reshape(H * C)
    nflat = attn_neigh[:, :, 0].T.reshape(H * C)
    sn = jnp.concatenate([sflat[:, None] * hsel, nflat[:, None] * hsel], axis=1)
    ksn = k2 @ sn

    g = pl.pallas_call(
        _gat_body,
        grid=(B // _NB,),
        in_specs=[
            pl.BlockSpec((T, _NB, N), lambda b: (0, b, 0)),
            pl.BlockSpec((N, N), lambda b: (0, 0)),
            pl.BlockSpec((H * C, T), lambda b: (0, 0)),
            pl.BlockSpec((T, 4), lambda b: (0, 0)),
            pl.BlockSpec((4, T), lambda b: (0, 0)),
            pl.BlockSpec((C, 1), lambda b: (0, 0)),
        ],
        out_specs=pl.BlockSpec((_NB, C, N), lambda b: (b, 0, 0)),
        out_shape=jax.ShapeDtypeStruct((B, C, N), jnp.bfloat16),
    )(x312, a.T, k2.T, ksn, ksn.T, gat_bias.reshape(C, 1))

    xflat = x312.reshape(T * B, N)
    h = pl.pallas_call(
        _lstm_body,
        out_shape=jax.ShapeDtypeStruct((B, LSTM_U), jnp.bfloat16),
    )(xflat, lstm_W.T, lstm_Uw, lstm_b.reshape(1, 4 * LSTM_U))

    g2 = jnp.transpose(g, (0, 2, 1)).reshape(B, N * C)
    z = jnp.concatenate([g2, h], axis=1)
    wt = dense_W.T

    out = pl.pallas_call(
        _dense_body,
        grid=(_NJB,),
        in_specs=[
            pl.BlockSpec((B, D_IN), lambda j: (0, 0)),
            pl.BlockSpec((_JBLK, D_IN), lambda j: (j, 0)),
            pl.BlockSpec((1, _JBLK), lambda j: (0, j)),
        ],
        out_specs=pl.BlockSpec((B, _JBLK), lambda j: (0, j)),
        out_shape=jax.ShapeDtypeStruct((B, D_OUT), f32),
    )(z, wt, dense_b.reshape(1, D_OUT))

    return out.reshape(B, OUT_STEPS, NUM_FEATURES)
